# baseline (speedup 1.0000x reference)
import functools

import jax
import jax.numpy as jnp
from jax import lax
from jax.experimental import pallas as pl
from jax.experimental.pallas import tpu as pltpu


def _pick_yb(y):
    best = y
    for cand in range(8, 257, 8):
        if y % cand == 0:
            best = cand
    return best if y % 8 == 0 else y


def _conv_body(e_ref, w_ref, b_ref, h_ref, *, L, K):
    F = w_ref.shape[2]
    acc = jnp.zeros((L, F), jnp.float32)
    for k in range(K):
        acc = acc + lax.dot_general(
            e_ref[0, k:k + L, :], w_ref[k],
            (((1,), (0,)), ((), ())),
            preferred_element_type=jnp.float32)
    h_ref[0] = jnp.tanh(acc + b_ref[0][None, :])


def _attn_body(h_ref, u_ref, fw_ref, fb_ref, t_ref, alpha_ref, yhat_ref,
               lsum_ref):
    h = h_ref[0].astype(jnp.bfloat16)
    u = u_ref[...].astype(jnp.bfloat16)
    fw = fw_ref[...].astype(jnp.bfloat16)
    sT = lax.dot_general(h, u, (((1,), (1,)), ((), ())),
                         preferred_element_type=jnp.float32)
    g = lax.dot_general(h, fw, (((1,), (1,)), ((), ())),
                        preferred_element_type=jnp.float32)
    smax = jnp.max(sT, axis=0, keepdims=True)
    p = jnp.exp(sT - smax)
    ssum = jnp.sum(p, axis=0, keepdims=True)
    alphaT = p / ssum
    alpha_ref[:, 0, 0, 0, :] = alphaT
    yh = jnp.sum(alphaT * g, axis=0, keepdims=True) + fb_ref[0]
    yhat_ref[0, 0] = yh
    t = t_ref[0, 0]
    bce = jnp.maximum(yh, 0.0) - yh * t + jnp.log1p(jnp.exp(-jnp.abs(yh)))
    lsum_ref[0, 0] = jnp.sum(bce, axis=1, keepdims=True)


def kernel(x, target, embed_W, conv_w, conv_b, U_w, final_w, final_b):
    B, L = x.shape
    D = embed_W.shape[1]
    F, _, K = conv_w.shape
    Y = U_w.shape[0]
    PAD = K // 2
    YB = _pick_yb(Y)
    NY = Y // YB

    e = embed_W[x]
    e_pad = jnp.pad(e, ((0, 0), (PAD, PAD), (0, 0)))
    w_t = jnp.transpose(conv_w, (2, 1, 0))

    conv_fn = pl.pallas_call(
        functools.partial(_conv_body, L=L, K=K),
        grid=(B,),
        in_specs=[
            pl.BlockSpec((1, L + K - 1, D), lambda b: (b, 0, 0)),
            pl.BlockSpec((K, D, F), lambda b: (0, 0, 0)),
            pl.BlockSpec((1, F), lambda b: (0, 0)),
        ],
        out_specs=pl.BlockSpec((1, L, F), lambda b: (b, 0, 0)),
        out_shape=jax.ShapeDtypeStruct((B, L, F), jnp.float32),
        compiler_params=pltpu.CompilerParams(
            dimension_semantics=("parallel",)),
    )
    h = conv_fn(e_pad, w_t, conv_b.reshape(1, F))

    attn_fn = pl.pallas_call(
        _attn_body,
        grid=(B, NY),
        in_specs=[
            pl.BlockSpec((1, L, F), lambda b, y: (b, 0, 0)),
            pl.BlockSpec((YB, F), lambda b, y: (y, 0)),
            pl.BlockSpec((YB, F), lambda b, y: (y, 0)),
            pl.BlockSpec((1, 1, YB), lambda b, y: (y, 0, 0)),
            pl.BlockSpec((1, 1, 1, YB), lambda b, y: (b, y, 0, 0)),
        ],
        out_specs=[
            pl.BlockSpec((L, 1, 1, 1, YB), lambda b, y: (0, b, y, 0, 0)),
            pl.BlockSpec((1, 1, 1, YB), lambda b, y: (b, y, 0, 0)),
            pl.BlockSpec((1, 1, 1, 1), lambda b, y: (b, y, 0, 0)),
        ],
        out_shape=[
            jax.ShapeDtypeStruct((L, B, NY, 1, YB), jnp.float32),
            jax.ShapeDtypeStruct((B, NY, 1, YB), jnp.float32),
            jax.ShapeDtypeStruct((B, NY, 1, 1), jnp.float32),
        ],
        compiler_params=pltpu.CompilerParams(
            dimension_semantics=("parallel", "arbitrary")),
    )
    alphaT, yhat4, lsums = attn_fn(h, U_w, final_w,
                                   final_b.reshape(NY, 1, YB),
                                   target.reshape(B, NY, 1, YB))
    alpha = jnp.transpose(alphaT.reshape(L, B, Y), (1, 2, 0))
    yhat = yhat4.reshape(B, Y)
    loss = jnp.sum(lsums) / (B * Y)
    return yhat, loss, alpha


# speedup vs baseline: 5.4076x; 5.4076x over previous
import functools

import jax
import jax.numpy as jnp
from jax import lax
from jax.experimental import pallas as pl
from jax.experimental.pallas import tpu as pltpu


def _conv_body(e_ref, w_ref, b_ref, h_ref, *, L, K):
    F = w_ref.shape[2]
    acc = jnp.zeros((L, F), jnp.float32)
    for k in range(K):
        acc = acc + lax.dot_general(
            e_ref[0, k:k + L, :], w_ref[k],
            (((1,), (0,)), ((), ())),
            preferred_element_type=jnp.float32)
    h_ref[0] = jnp.tanh(acc + b_ref[0][None, :]).astype(jnp.bfloat16)


def _attn_body(h_ref, u_ref, fw_ref, fb_ref, t_ref, alpha_ref, yhat_ref,
               lsum_ref, *, B, Y, YB):
    ny = pl.program_id(0)
    u = u_ref[...]
    fw = fw_ref[...]
    fb = fb_ref[...]
    col = lax.broadcasted_iota(jnp.int32, (1, YB), 1) + ny * YB
    valid = col < Y
    loss_acc = jnp.zeros((1, 1), jnp.float32)
    for b in range(B):
        h = h_ref[b]
        sT = lax.dot_general(h, u, (((1,), (1,)), ((), ())),
                             preferred_element_type=jnp.float32)
        g = lax.dot_general(h, fw, (((1,), (1,)), ((), ())),
                            preferred_element_type=jnp.float32)
        smax = jnp.max(sT, axis=0, keepdims=True)
        p = jnp.exp(sT - smax)
        ssum = jnp.sum(p, axis=0, keepdims=True)
        alphaT = p / ssum
        alpha_ref[:, b, :] = alphaT
        yh = jnp.sum(alphaT * g, axis=0, keepdims=True) + fb
        yhat_ref[b:b + 1, :] = yh
        t = t_ref[b:b + 1, :]
        bce = jnp.maximum(yh, 0.0) - yh * t + jnp.log1p(jnp.exp(-jnp.abs(yh)))
        bce = jnp.where(valid, bce, 0.0)
        loss_acc = loss_acc + jnp.sum(bce, axis=1, keepdims=True)
    lsum_ref[0] = loss_acc


def kernel(x, target, embed_W, conv_w, conv_b, U_w, final_w, final_b):
    B, L = x.shape
    D = embed_W.shape[1]
    F, _, K = conv_w.shape
    Y = U_w.shape[0]
    PAD = K // 2
    YB = 128 if Y >= 128 else 16
    NY = -(-Y // YB)

    e = embed_W[x]
    e_pad = jnp.pad(e, ((0, 0), (PAD, PAD), (0, 0)))
    w_t = jnp.transpose(conv_w, (2, 1, 0))

    conv_fn = pl.pallas_call(
        functools.partial(_conv_body, L=L, K=K),
        grid=(B,),
        in_specs=[
            pl.BlockSpec((1, L + K - 1, D), lambda b: (b, 0, 0)),
            pl.BlockSpec((K, D, F), lambda b: (0, 0, 0)),
            pl.BlockSpec((1, F), lambda b: (0, 0)),
        ],
        out_specs=pl.BlockSpec((1, L, F), lambda b: (b, 0, 0)),
        out_shape=jax.ShapeDtypeStruct((B, L, F), jnp.bfloat16),
        compiler_params=pltpu.CompilerParams(
            dimension_semantics=("parallel",)),
    )
    h = conv_fn(e_pad, w_t, conv_b.reshape(1, F))

    attn_fn = pl.pallas_call(
        functools.partial(_attn_body, B=B, Y=Y, YB=YB),
        grid=(NY,),
        in_specs=[
            pl.BlockSpec((B, L, F), lambda y: (0, 0, 0)),
            pl.BlockSpec((YB, F), lambda y: (y, 0)),
            pl.BlockSpec((YB, F), lambda y: (y, 0)),
            pl.BlockSpec((1, YB), lambda y: (0, y)),
            pl.BlockSpec((B, YB), lambda y: (0, y)),
        ],
        out_specs=[
            pl.BlockSpec((L, B, YB), lambda y: (0, 0, y)),
            pl.BlockSpec((B, YB), lambda y: (0, y)),
            pl.BlockSpec((1, 1, 1), lambda y: (y, 0, 0)),
        ],
        out_shape=[
            jax.ShapeDtypeStruct((L, B, Y), jnp.float32),
            jax.ShapeDtypeStruct((B, Y), jnp.float32),
            jax.ShapeDtypeStruct((NY, 1, 1), jnp.float32),
        ],
        compiler_params=pltpu.CompilerParams(
            dimension_semantics=("parallel",),
            vmem_limit_bytes=56 * 2**20),
    )
    alphaT, yhat, lsums = attn_fn(h, U_w.astype(jnp.bfloat16),
                                  final_w.astype(jnp.bfloat16),
                                  final_b.reshape(1, Y), target)
    alpha = jnp.transpose(alphaT, (1, 2, 0))
    loss = jnp.sum(lsums) / (B * Y)
    return yhat, loss, alpha


# speedup vs baseline: 5.5542x; 1.0271x over previous
import functools

import jax
import jax.numpy as jnp
from jax import lax
from jax.experimental import pallas as pl
from jax.experimental.pallas import tpu as pltpu


def _conv_body(x_ref, tab_ref, w_ref, b_ref, h_ref, e_ref, *, L, K, G):
    F = w_ref.shape[2]
    PAD = K // 2
    e_ref[0:PAD, 0, :] = jnp.zeros((PAD, e_ref.shape[2]), jnp.float32)
    e_ref[L + PAD:L + 2 * PAD, 0, :] = jnp.zeros((PAD, e_ref.shape[2]),
                                                 jnp.float32)

    def body(i, _):
        for j in range(G):
            l = i * G + j
            e_ref[l + PAD, 0, :] = tab_ref[x_ref[0, 0, l], 0, :]
        return ()
    lax.fori_loop(0, L // G, body, (), unroll=2)

    e = e_ref[:, 0, :]
    acc = jnp.zeros((L, F), jnp.float32)
    for k in range(K):
        acc = acc + lax.dot_general(
            e[k:k + L, :], w_ref[k],
            (((1,), (0,)), ((), ())),
            preferred_element_type=jnp.float32)
    h_ref[0] = jnp.tanh(acc + b_ref[0][None, :]).astype(jnp.bfloat16)


def _attn_body(h_ref, u_ref, fw_ref, fb_ref, t_ref, alpha_ref, yhat_ref,
               lsum_ref, *, B, Y, YB):
    ny = pl.program_id(0)
    u = u_ref[...]
    fw = fw_ref[...]
    fb = fb_ref[...]
    col = lax.broadcasted_iota(jnp.int32, (1, YB), 1) + ny * YB
    valid = col < Y
    loss_acc = jnp.zeros((1, 1), jnp.float32)
    for b in range(B):
        h = h_ref[b]
        sT = lax.dot_general(h, u, (((1,), (1,)), ((), ())),
                             preferred_element_type=jnp.float32)
        g = lax.dot_general(h, fw, (((1,), (1,)), ((), ())),
                            preferred_element_type=jnp.float32)
        smax = jnp.max(sT, axis=0, keepdims=True)
        p = jnp.exp(sT - smax)
        ssum = jnp.sum(p, axis=0, keepdims=True)
        alphaT = p / ssum
        alpha_ref[:, b, :] = alphaT
        yh = jnp.sum(alphaT * g, axis=0, keepdims=True) + fb
        yhat_ref[b:b + 1, :] = yh
        t = t_ref[b:b + 1, :]
        bce = jnp.maximum(yh, 0.0) - yh * t + jnp.log1p(jnp.exp(-jnp.abs(yh)))
        bce = jnp.where(valid, bce, 0.0)
        loss_acc = loss_acc + jnp.sum(bce, axis=1, keepdims=True)
    lsum_ref[0] = loss_acc


def kernel(x, target, embed_W, conv_w, conv_b, U_w, final_w, final_b):
    B, L = x.shape
    D = embed_W.shape[1]
    F, _, K = conv_w.shape
    Y = U_w.shape[0]
    PAD = K // 2
    YB = 128 if Y >= 128 else 16
    NY = -(-Y // YB)

    V = embed_W.shape[0]
    w_t = jnp.transpose(conv_w, (2, 1, 0))
    G = 4 if L % 4 == 0 else 1
    conv_fn = pl.pallas_call(
        functools.partial(_conv_body, L=L, K=K, G=G),
        grid=(B,),
        in_specs=[
            pl.BlockSpec((1, 1, L), lambda b: (b, 0, 0),
                         memory_space=pltpu.SMEM),
            pl.BlockSpec((V, 1, D), lambda b: (0, 0, 0)),
            pl.BlockSpec((K, D, F), lambda b: (0, 0, 0)),
            pl.BlockSpec((1, F), lambda b: (0, 0)),
        ],
        out_specs=pl.BlockSpec((1, L, F), lambda b: (b, 0, 0)),
        out_shape=jax.ShapeDtypeStruct((B, L, F), jnp.bfloat16),
        scratch_shapes=[pltpu.VMEM((L + K - 1, 1, D), jnp.float32)],
        compiler_params=pltpu.CompilerParams(
            dimension_semantics=("arbitrary",),
            vmem_limit_bytes=56 * 2**20),
    )
    h = conv_fn(x.reshape(B, 1, L), embed_W.reshape(V, 1, D), w_t,
                conv_b.reshape(1, F))

    attn_fn = pl.pallas_call(
        functools.partial(_attn_body, B=B, Y=Y, YB=YB),
        grid=(NY,),
        in_specs=[
            pl.BlockSpec((B, L, F), lambda y: (0, 0, 0)),
            pl.BlockSpec((YB, F), lambda y: (y, 0)),
            pl.BlockSpec((YB, F), lambda y: (y, 0)),
            pl.BlockSpec((1, YB), lambda y: (0, y)),
            pl.BlockSpec((B, YB), lambda y: (0, y)),
        ],
        out_specs=[
            pl.BlockSpec((L, B, YB), lambda y: (0, 0, y)),
            pl.BlockSpec((B, YB), lambda y: (0, y)),
            pl.BlockSpec((1, 1, 1), lambda y: (y, 0, 0)),
        ],
        out_shape=[
            jax.ShapeDtypeStruct((L, B, Y), jnp.float32),
            jax.ShapeDtypeStruct((B, Y), jnp.float32),
            jax.ShapeDtypeStruct((NY, 1, 1), jnp.float32),
        ],
        compiler_params=pltpu.CompilerParams(
            dimension_semantics=("arbitrary",),
            vmem_limit_bytes=56 * 2**20),
    )
    alphaT, yhat, lsums = attn_fn(h, U_w.astype(jnp.bfloat16),
                                  final_w.astype(jnp.bfloat16),
                                  final_b.reshape(1, Y), target)
    alpha = jnp.transpose(alphaT, (1, 2, 0))
    loss = jnp.sum(lsums) / (B * Y)
    return yhat, loss, alpha
